# baseline (speedup 1.0000x reference)
import math

import jax
import jax.numpy as jnp
from jax import lax
from jax.experimental import pallas as pl
from jax.experimental.pallas import tpu as pltpu

G_DIM = 768
H0 = 300
H = 150
H1 = 150
H2 = 150
L = 512
WP = 5
WF = 5
B = 8
S = 16

CHUNK = 32
NT = L // CHUNK

_F32 = jnp.float32


def _shift(x, d):
    if d == 0:
        return x
    n = x.shape[0]
    k = d % n
    return jnp.concatenate([x[k:], x[:k]], axis=0)


def _proj_body(xa_ref, xb_ref, wds_ref, bds_ref, wif_ref, bif_ref, wib_ref,
               bib_ref, of_ref, ob_ref):
    x = jnp.concatenate(
        [xa_ref[...].reshape(B * CHUNK, G_DIM),
         xb_ref[...].reshape(B * CHUNK, G_DIM)], axis=0)
    h0 = jnp.dot(x, wds_ref[...], preferred_element_type=_F32) + bds_ref[...]
    xf = jnp.dot(h0, wif_ref[...], preferred_element_type=_F32) + bif_ref[...]
    xb = jnp.dot(h0, wib_ref[...], preferred_element_type=_F32) + bib_ref[...]
    of_ref[...] = xf.reshape(S, CHUNK, 3 * H)
    ob_ref[...] = xb.reshape(S, CHUNK, 3 * H)


def _scan_body(xwf_ref, xwb_ref, whf_ref, whb_ref, bhn_ref,
               ff_ref, fb_ref, h_ref):
    g = pl.program_id(0)

    @pl.when(g == 0)
    def _init():
        h_ref[...] = jnp.zeros((S, 2 * H), _F32)

    bhn = bhn_ref[...]

    def gru(xw, gh, h, bhn_c):
        r = jax.nn.sigmoid(xw[:, :H] + gh[:, :H])
        z = jax.nn.sigmoid(xw[:, H:2 * H] + gh[:, H:2 * H])
        n = jnp.tanh(xw[:, 2 * H:] + r * (gh[:, 2 * H:] + bhn_c))
        return n + z * (h - n)

    def step(k, h):
        kk = CHUNK - 1 - k
        xwf = xwf_ref[:, k, :]
        xwb = xwb_ref[:, kk, :]
        ghf = jnp.dot(h[:, :H], whf_ref[...], preferred_element_type=_F32)
        ghb = jnp.dot(h[:, H:], whb_ref[...], preferred_element_type=_F32)
        h_f = gru(xwf, ghf, h[:, :H], bhn[:, :H])
        h_b = gru(xwb, ghb, h[:, H:], bhn[:, H:])
        ff_ref[:, k, :] = h_f
        fb_ref[:, kk, :] = h_b
        return jnp.concatenate([h_f, h_b], axis=1)

    h_ref[...] = lax.fori_loop(0, CHUNK, step, h_ref[...])


def _gcn_body(ff_ref, fb_ref, spk_ref, len_ref,
              watt_ref, wrel_ref, wroot_ref, brg_ref,
              wg1_ref, wg2_ref, bg_ref, ws1_ref, bs1_ref, ws2_ref, bs2_ref,
              out_ref):
    f = jnp.concatenate([ff_ref[0], fb_ref[0]], axis=1)
    spk = spk_ref[0]
    lens = jnp.maximum(len_ref[0], 1)
    t = lax.broadcasted_iota(jnp.int32, (L, 1), 0)
    nv = t < lens

    xatt = jnp.dot(f, watt_ref[...], preferred_element_type=_F32)
    inv_sqrt = 1.0 / math.sqrt(float(H0))
    scs = []
    evs = []
    for d in range(-WP, WF + 1):
        xs = _shift(xatt, d)
        scs.append(jnp.sum(f * xs, axis=1, keepdims=True) * inv_sqrt)
        evs.append(
            (nv & (t + d >= 0) & (t + d <= lens - 1)).astype(_F32))
    sc = jnp.concatenate(scs, axis=1)
    ev = jnp.concatenate(evs, axis=1) > 0.0
    m = jnp.max(jnp.where(ev, sc, -1e30), axis=1, keepdims=True)
    e = jnp.where(ev, jnp.exp(sc - m), 0.0)
    ssum = jnp.sum(e, axis=1, keepdims=True)
    norm = e / (ssum + 1e-9)

    wrel = wrel_ref[...]
    src1 = spk == 1
    offs = list(range(-WP, WF + 1))
    msg = jnp.zeros((L, H1), _F32)
    for c in (0, 1):
        u0 = jnp.where(
            src1,
            jnp.dot(f, wrel[4 + c], preferred_element_type=_F32),
            jnp.dot(f, wrel[c], preferred_element_type=_F32))
        u1 = jnp.where(
            src1,
            jnp.dot(f, wrel[6 + c], preferred_element_type=_F32),
            jnp.dot(f, wrel[2 + c], preferred_element_type=_F32))
        for di, d in enumerate(offs):
            if (0 if d < 0 else 1) != c:
                continue
            row = jnp.where(src1, _shift(u1, d), _shift(u0, d))
            msg = msg + norm[:, di:di + 1] * row
    x1 = msg + jnp.dot(f, wroot_ref[...], preferred_element_type=_F32) + brg_ref[...]

    agg2 = jnp.zeros((L, H1), _F32)
    for di, d in enumerate(offs):
        agg2 = agg2 + norm[:, di:di + 1] * _shift(x1, d)
    x2 = (jnp.dot(agg2, wg2_ref[...], preferred_element_type=_F32)
          + jnp.dot(x1, wg1_ref[...], preferred_element_type=_F32)
          + bg_ref[...])

    inv_l = nv.astype(_F32) / lens.astype(_F32)
    pooled_f = jnp.sum(f * inv_l, axis=0, keepdims=True)
    pooled_x = jnp.sum(x2 * inv_l, axis=0, keepdims=True)
    pooled = jnp.concatenate([pooled_f, pooled_x], axis=1)
    h = jnp.maximum(
        jnp.dot(pooled, ws1_ref[...], preferred_element_type=_F32) + bs1_ref[...],
        0.0)
    out_ref[0] = jnp.dot(h, ws2_ref[...], preferred_element_type=_F32) + bs2_ref[...]


def kernel(a_text_tensor, a_text_len_tensor, a_speaker_tensor, b_text_tensor,
           b_text_len_tensor, b_speaker_tensor, W_ds, b_ds, Wi_f, Wh_f, bi_f,
           bh_f, Wi_b, Wh_b, bi_b, bh_b, W_att, W_rel, W_root, b_rg, W_g1,
           W_g2, b_g, W_s1, b_s1, W_s2, b_s2):
    spk = jnp.concatenate([a_speaker_tensor, b_speaker_tensor],
                          axis=0).astype(jnp.int32).reshape(S, L, 1)
    lens = jnp.concatenate([a_text_len_tensor, b_text_len_tensor],
                           axis=0).astype(jnp.int32).reshape(S, 1, 1)

    zero_h = jnp.zeros((H,), _F32)
    bi_f_eff = bi_f + jnp.concatenate([bh_f[:2 * H], zero_h])
    bi_b_eff = bi_b + jnp.concatenate([bh_b[:2 * H], zero_h])
    bhn = jnp.concatenate([bh_f[2 * H:], bh_b[2 * H:]]).reshape(1, 2 * H)

    full = lambda shape: pl.BlockSpec(shape, lambda *_: (0,) * len(shape))

    xw_f, xw_b = pl.pallas_call(
        _proj_body,
        grid=(NT,),
        in_specs=[
            pl.BlockSpec((B, CHUNK, G_DIM), lambda g: (0, g, 0)),
            pl.BlockSpec((B, CHUNK, G_DIM), lambda g: (0, g, 0)),
            full((G_DIM, H0)),
            full((1, H0)),
            full((H0, 3 * H)),
            full((1, 3 * H)),
            full((H0, 3 * H)),
            full((1, 3 * H)),
        ],
        out_specs=[
            pl.BlockSpec((S, CHUNK, 3 * H), lambda g: (0, g, 0)),
            pl.BlockSpec((S, CHUNK, 3 * H), lambda g: (0, g, 0)),
        ],
        out_shape=[
            jax.ShapeDtypeStruct((S, L, 3 * H), _F32),
            jax.ShapeDtypeStruct((S, L, 3 * H), _F32),
        ],
        compiler_params=pltpu.CompilerParams(
            dimension_semantics=("parallel",)),
    )(a_text_tensor, b_text_tensor, W_ds, b_ds.reshape(1, H0),
      Wi_f, bi_f_eff.reshape(1, 3 * H), Wi_b, bi_b_eff.reshape(1, 3 * H))

    f_fwd, f_bwd = pl.pallas_call(
        _scan_body,
        grid=(NT,),
        in_specs=[
            pl.BlockSpec((S, CHUNK, 3 * H), lambda g: (0, g, 0)),
            pl.BlockSpec((S, CHUNK, 3 * H), lambda g: (0, NT - 1 - g, 0)),
            full((H, 3 * H)),
            full((H, 3 * H)),
            full((1, 2 * H)),
        ],
        out_specs=[
            pl.BlockSpec((S, CHUNK, H), lambda g: (0, g, 0)),
            pl.BlockSpec((S, CHUNK, H), lambda g: (0, NT - 1 - g, 0)),
        ],
        out_shape=[
            jax.ShapeDtypeStruct((S, L, H), _F32),
            jax.ShapeDtypeStruct((S, L, H), _F32),
        ],
        scratch_shapes=[pltpu.VMEM((S, 2 * H), _F32)],
        compiler_params=pltpu.CompilerParams(
            dimension_semantics=("arbitrary",)),
    )(xw_f, xw_b, Wh_f, Wh_b, bhn)

    coh = pl.pallas_call(
        _gcn_body,
        grid=(S,),
        in_specs=[
            pl.BlockSpec((1, L, H), lambda i: (i, 0, 0)),
            pl.BlockSpec((1, L, H), lambda i: (i, 0, 0)),
            pl.BlockSpec((1, L, 1), lambda i: (i, 0, 0)),
            pl.BlockSpec((1, 1, 1), lambda i: (i, 0, 0)),
            full((H0, H0)),
            full((8, H0, H1)),
            full((H0, H1)),
            full((1, H1)),
            full((H1, H2)),
            full((H1, H2)),
            full((1, H2)),
            full((H0 + H2, H1)),
            full((1, H1)),
            full((H1, 1)),
            full((1, 1)),
        ],
        out_specs=pl.BlockSpec((1, 1, 1), lambda i: (i, 0, 0)),
        out_shape=jax.ShapeDtypeStruct((S, 1, 1), _F32),
        compiler_params=pltpu.CompilerParams(
            dimension_semantics=("parallel",)),
    )(f_fwd, f_bwd, spk, lens, W_att, W_rel, W_root, b_rg.reshape(1, H1),
      W_g1, W_g2, b_g.reshape(1, H2), W_s1, b_s1.reshape(1, H1),
      W_s2, b_s2.reshape(1, 1))

    coh = coh.reshape(S)
    a_coh = coh[:8]
    b_coh = coh[8:]
    rst = (b_coh > a_coh).astype(jnp.int32)
    return (rst, a_coh)


# speedup vs baseline: 1.3003x; 1.3003x over previous
import math

import jax
import jax.numpy as jnp
from jax import lax
from jax.experimental import pallas as pl
from jax.experimental.pallas import tpu as pltpu

G_DIM = 768
H0 = 300
H = 150
H1 = 150
H2 = 150
L = 512
WP = 5
WF = 5
B = 8
S = 16

CHUNK = 32
NT = L // CHUNK

_F32 = jnp.float32


def _shift(x, d):
    if d == 0:
        return x
    n = x.shape[0]
    k = d % n
    return jnp.concatenate([x[k:], x[:k]], axis=0)


def _proj_body(xa_ref, xb_ref, wds_ref, bds_ref, wif_ref, bif_ref, wib_ref,
               bib_ref, of_ref, ob_ref):
    x = jnp.concatenate(
        [xa_ref[...].reshape(B * CHUNK, G_DIM),
         xb_ref[...].reshape(B * CHUNK, G_DIM)], axis=0)
    h0 = jnp.dot(x, wds_ref[...], preferred_element_type=_F32) + bds_ref[...]
    xf = jnp.dot(h0, wif_ref[...], preferred_element_type=_F32) + bif_ref[...]
    xb = jnp.dot(h0, wib_ref[...], preferred_element_type=_F32) + bib_ref[...]
    of_ref[...] = xf.reshape(S, CHUNK, 3 * H)
    ob_ref[...] = xb.reshape(S, CHUNK, 3 * H)


def _scan_body(xwf_ref, xwb_ref, whf_ref, whb_ref, bhn_ref,
               ff_ref, fb_ref, h_ref):
    g = pl.program_id(0)

    @pl.when(g == 0)
    def _init():
        h_ref[...] = jnp.zeros((S, 2 * H), _F32)

    bhn = bhn_ref[...]

    def gru(xw, gh, h, bhn_c):
        r = jax.nn.sigmoid(xw[:, :H] + gh[:, :H])
        z = jax.nn.sigmoid(xw[:, H:2 * H] + gh[:, H:2 * H])
        n = jnp.tanh(xw[:, 2 * H:] + r * (gh[:, 2 * H:] + bhn_c))
        return n + z * (h - n)

    def step(k, carry):
        h_f, h_b = carry
        kk = CHUNK - 1 - k
        xwf = xwf_ref[:, k, :]
        xwb = xwb_ref[:, kk, :]
        ghf = jnp.dot(h_f, whf_ref[...], preferred_element_type=_F32)
        ghb = jnp.dot(h_b, whb_ref[...], preferred_element_type=_F32)
        h_f2 = gru(xwf, ghf, h_f, bhn[:, :H])
        h_b2 = gru(xwb, ghb, h_b, bhn[:, H:])
        ff_ref[:, k, :] = h_f2
        fb_ref[:, kk, :] = h_b2
        return (h_f2, h_b2)

    h0 = h_ref[...]
    hf, hb = lax.fori_loop(0, CHUNK, step, (h0[:, :H], h0[:, H:]))
    h_ref[...] = jnp.concatenate([hf, hb], axis=1)


def _gcn_body(ff_ref, fb_ref, spk_ref, len_ref,
              watt_ref, wrel_ref, wroot_ref, brg_ref,
              wg1_ref, wg2_ref, bg_ref, ws1_ref, bs1_ref, ws2_ref, bs2_ref,
              out_ref):
    f = jnp.concatenate([ff_ref[0], fb_ref[0]], axis=1)
    spk = spk_ref[0]
    lens = jnp.maximum(len_ref[0], 1)
    t = lax.broadcasted_iota(jnp.int32, (L, 1), 0)
    nv = t < lens

    xatt = jnp.dot(f, watt_ref[...], preferred_element_type=_F32)
    inv_sqrt = 1.0 / math.sqrt(float(H0))
    scs = []
    evs = []
    for d in range(-WP, WF + 1):
        xs = _shift(xatt, d)
        scs.append(jnp.sum(f * xs, axis=1, keepdims=True) * inv_sqrt)
        evs.append(
            (nv & (t + d >= 0) & (t + d <= lens - 1)).astype(_F32))
    sc = jnp.concatenate(scs, axis=1)
    ev = jnp.concatenate(evs, axis=1) > 0.0
    m = jnp.max(jnp.where(ev, sc, -1e30), axis=1, keepdims=True)
    e = jnp.where(ev, jnp.exp(sc - m), 0.0)
    ssum = jnp.sum(e, axis=1, keepdims=True)
    norm = e / (ssum + 1e-9)

    wrel = wrel_ref[...]
    src1 = spk == 1
    offs = list(range(-WP, WF + 1))
    msg = jnp.zeros((L, H1), _F32)
    for c in (0, 1):
        u0 = jnp.where(
            src1,
            jnp.dot(f, wrel[4 + c], preferred_element_type=_F32),
            jnp.dot(f, wrel[c], preferred_element_type=_F32))
        u1 = jnp.where(
            src1,
            jnp.dot(f, wrel[6 + c], preferred_element_type=_F32),
            jnp.dot(f, wrel[2 + c], preferred_element_type=_F32))
        for di, d in enumerate(offs):
            if (0 if d < 0 else 1) != c:
                continue
            row = jnp.where(src1, _shift(u1, d), _shift(u0, d))
            msg = msg + norm[:, di:di + 1] * row
    x1 = msg + jnp.dot(f, wroot_ref[...], preferred_element_type=_F32) + brg_ref[...]

    agg2 = jnp.zeros((L, H1), _F32)
    for di, d in enumerate(offs):
        agg2 = agg2 + norm[:, di:di + 1] * _shift(x1, d)
    x2 = (jnp.dot(agg2, wg2_ref[...], preferred_element_type=_F32)
          + jnp.dot(x1, wg1_ref[...], preferred_element_type=_F32)
          + bg_ref[...])

    inv_l = nv.astype(_F32) / lens.astype(_F32)
    pooled_f = jnp.sum(f * inv_l, axis=0, keepdims=True)
    pooled_x = jnp.sum(x2 * inv_l, axis=0, keepdims=True)
    pooled = jnp.concatenate([pooled_f, pooled_x], axis=1)
    h = jnp.maximum(
        jnp.dot(pooled, ws1_ref[...], preferred_element_type=_F32) + bs1_ref[...],
        0.0)
    out_ref[0] = jnp.dot(h, ws2_ref[...], preferred_element_type=_F32) + bs2_ref[...]


def kernel(a_text_tensor, a_text_len_tensor, a_speaker_tensor, b_text_tensor,
           b_text_len_tensor, b_speaker_tensor, W_ds, b_ds, Wi_f, Wh_f, bi_f,
           bh_f, Wi_b, Wh_b, bi_b, bh_b, W_att, W_rel, W_root, b_rg, W_g1,
           W_g2, b_g, W_s1, b_s1, W_s2, b_s2):
    spk = jnp.concatenate([a_speaker_tensor, b_speaker_tensor],
                          axis=0).astype(jnp.int32).reshape(S, L, 1)
    lens = jnp.concatenate([a_text_len_tensor, b_text_len_tensor],
                           axis=0).astype(jnp.int32).reshape(S, 1, 1)

    zero_h = jnp.zeros((H,), _F32)
    bi_f_eff = bi_f + jnp.concatenate([bh_f[:2 * H], zero_h])
    bi_b_eff = bi_b + jnp.concatenate([bh_b[:2 * H], zero_h])
    bhn = jnp.concatenate([bh_f[2 * H:], bh_b[2 * H:]]).reshape(1, 2 * H)

    full = lambda shape: pl.BlockSpec(shape, lambda *_: (0,) * len(shape))

    xw_f, xw_b = pl.pallas_call(
        _proj_body,
        grid=(NT,),
        in_specs=[
            pl.BlockSpec((B, CHUNK, G_DIM), lambda g: (0, g, 0)),
            pl.BlockSpec((B, CHUNK, G_DIM), lambda g: (0, g, 0)),
            full((G_DIM, H0)),
            full((1, H0)),
            full((H0, 3 * H)),
            full((1, 3 * H)),
            full((H0, 3 * H)),
            full((1, 3 * H)),
        ],
        out_specs=[
            pl.BlockSpec((S, CHUNK, 3 * H), lambda g: (0, g, 0)),
            pl.BlockSpec((S, CHUNK, 3 * H), lambda g: (0, g, 0)),
        ],
        out_shape=[
            jax.ShapeDtypeStruct((S, L, 3 * H), _F32),
            jax.ShapeDtypeStruct((S, L, 3 * H), _F32),
        ],
        compiler_params=pltpu.CompilerParams(
            dimension_semantics=("parallel",)),
    )(a_text_tensor, b_text_tensor, W_ds, b_ds.reshape(1, H0),
      Wi_f, bi_f_eff.reshape(1, 3 * H), Wi_b, bi_b_eff.reshape(1, 3 * H))

    f_fwd, f_bwd = pl.pallas_call(
        _scan_body,
        grid=(NT,),
        in_specs=[
            pl.BlockSpec((S, CHUNK, 3 * H), lambda g: (0, g, 0)),
            pl.BlockSpec((S, CHUNK, 3 * H), lambda g: (0, NT - 1 - g, 0)),
            full((H, 3 * H)),
            full((H, 3 * H)),
            full((1, 2 * H)),
        ],
        out_specs=[
            pl.BlockSpec((S, CHUNK, H), lambda g: (0, g, 0)),
            pl.BlockSpec((S, CHUNK, H), lambda g: (0, NT - 1 - g, 0)),
        ],
        out_shape=[
            jax.ShapeDtypeStruct((S, L, H), _F32),
            jax.ShapeDtypeStruct((S, L, H), _F32),
        ],
        scratch_shapes=[pltpu.VMEM((S, 2 * H), _F32)],
        compiler_params=pltpu.CompilerParams(
            dimension_semantics=("arbitrary",)),
    )(xw_f, xw_b, Wh_f, Wh_b, bhn)

    coh = pl.pallas_call(
        _gcn_body,
        grid=(S,),
        in_specs=[
            pl.BlockSpec((1, L, H), lambda i: (i, 0, 0)),
            pl.BlockSpec((1, L, H), lambda i: (i, 0, 0)),
            pl.BlockSpec((1, L, 1), lambda i: (i, 0, 0)),
            pl.BlockSpec((1, 1, 1), lambda i: (i, 0, 0)),
            full((H0, H0)),
            full((8, H0, H1)),
            full((H0, H1)),
            full((1, H1)),
            full((H1, H2)),
            full((H1, H2)),
            full((1, H2)),
            full((H0 + H2, H1)),
            full((1, H1)),
            full((H1, 1)),
            full((1, 1)),
        ],
        out_specs=pl.BlockSpec((1, 1, 1), lambda i: (i, 0, 0)),
        out_shape=jax.ShapeDtypeStruct((S, 1, 1), _F32),
        compiler_params=pltpu.CompilerParams(
            dimension_semantics=("parallel",)),
    )(f_fwd, f_bwd, spk, lens, W_att, W_rel, W_root, b_rg.reshape(1, H1),
      W_g1, W_g2, b_g.reshape(1, H2), W_s1, b_s1.reshape(1, H1),
      W_s2, b_s2.reshape(1, 1))

    coh = coh.reshape(S)
    a_coh = coh[:8]
    b_coh = coh[8:]
    rst = (b_coh > a_coh).astype(jnp.int32)
    return (rst, a_coh)
